# Initial kernel scaffold; baseline (speedup 1.0000x reference)
import jax
import jax.numpy as jnp
from jax.experimental import pallas as pl


def kernel(pieces_ids, color_ids, indexes, position_emb, piece_emb, color_emb):
    raise NotImplementedError("write your pallas kernel here")


# speedup vs baseline: 16.3847x; 16.3847x over previous
import functools

import jax
import jax.numpy as jnp
from jax import lax
from jax.experimental import pallas as pl
from jax.experimental.pallas import tpu as pltpu
from jax.experimental.pallas import tpu_sc as plsc

D = 128
NPOS, NPC, NCOL = 64, 6, 2
NFUSED = NPOS * NPC * NCOL

B, L = 16384, 32
TOK = B * L
NC, NS = 2, 16
NW = NC * NS
TPW = TOK // NW
CHUNK = 128
NCHUNK = TPW // CHUNK
ROWS = TOK // D
NBUF = 4


def _fuse_tables_kernel(pos_ref, pc_ref, col_ref, out_ref):
    pos = pos_ref[:]
    for m in range(NPC * NCOL):
        p, c = m // NCOL, m % NCOL
        out_ref[pl.ds(m * NPOS, NPOS), :] = (
            pos + pc_ref[pl.ds(p, 1), :] + col_ref[pl.ds(c, 1), :]
        )


def _fuse_tables(pos, pc, col):
    return pl.pallas_call(
        _fuse_tables_kernel,
        out_shape=jax.ShapeDtypeStruct((NFUSED, D), jnp.float32),
    )(pos, pc, col)


def _fidx_kernel(p_ref, c_ref, i_ref, o_ref):
    o_ref[:] = (p_ref[:] * NCOL + c_ref[:]) * NPOS + i_ref[:]


def _fused_index(p2, c2, i2):
    grid = 8
    blk = ROWS // grid
    return pl.pallas_call(
        _fidx_kernel,
        grid=(grid,),
        in_specs=[pl.BlockSpec((blk, D), lambda g: (g, 0))] * 3,
        out_specs=pl.BlockSpec((blk, D), lambda g: (g, 0)),
        out_shape=jax.ShapeDtypeStruct((ROWS, D), jnp.int32),
    )(p2, c2, i2)


def _sc_gather_body(tab_hbm, fidx_hbm, out_hbm, fidx_v, rows, semg, semo):
    wid = lax.axis_index("s") * NC + lax.axis_index("c")
    row0 = wid * (TPW // D)
    tok0 = wid * TPW

    pltpu.sync_copy(fidx_hbm.at[pl.ds(row0, TPW // D)], fidx_v)

    def gather(g, b):
        pltpu.async_copy(tab_hbm.at[fidx_v.at[g]], rows[b], semg[b])

    def put(g, b):
        pltpu.async_copy(rows[b], out_hbm.at[pl.ds(tok0 + g * CHUNK, CHUNK)],
                         semo[b])

    def wait_gather(g, b):
        pltpu.make_async_copy(tab_hbm.at[fidx_v.at[g]], rows[b], semg[b]).wait()

    def wait_put(g, b):
        pltpu.make_async_copy(rows[b],
                              out_hbm.at[pl.ds(tok0 + g * CHUNK, CHUNK)],
                              semo[b]).wait()

    gather(0, 0)
    gather(1, 1)

    def step(h, _):
        for b in range(NBUF):
            g = h * NBUF + b
            g2 = g + 2
            b2 = (b + 2) % NBUF

            @pl.when(g2 <= NCHUNK - 1)
            def _issue():
                @pl.when(g >= 2)
                def _free():
                    wait_put(g - 2, b2)
                gather(g2, b2)

            wait_gather(g, b)
            put(g, b)
        return _

    lax.fori_loop(0, NCHUNK // NBUF, step, None)

    for b in range(NBUF):
        wait_put(NCHUNK - NBUF + b, b)


def _sc_gather(tab, fidx2):
    mesh = plsc.VectorSubcoreMesh(core_axis_name="c", subcore_axis_name="s")
    f = functools.partial(
        pl.kernel,
        out_type=jax.ShapeDtypeStruct((TOK, D), jnp.float32),
        mesh=mesh,
        scratch_types=[
            pltpu.VMEM((TPW // D, D), jnp.int32),
            [pltpu.VMEM((CHUNK, D), jnp.float32) for _ in range(NBUF)],
            [pltpu.SemaphoreType.DMA for _ in range(NBUF)],
            [pltpu.SemaphoreType.DMA for _ in range(NBUF)],
        ],
    )(_sc_gather_body)
    return f(tab, fidx2)


def kernel(pieces_ids, color_ids, indexes, position_emb, piece_emb, color_emb):
    tab = _fuse_tables(position_emb, piece_emb, color_emb)
    p2 = pieces_ids.astype(jnp.int32).reshape(ROWS, D)
    c2 = color_ids.astype(jnp.int32).reshape(ROWS, D)
    i2 = indexes.astype(jnp.int32).reshape(ROWS, D)
    fidx2 = _fused_index(p2, c2, i2)
    out = _sc_gather(tab, fidx2)
    return out.reshape(B, L, D)


# speedup vs baseline: 37.0608x; 2.2619x over previous
import functools

import jax
import jax.numpy as jnp
from jax import lax
from jax.experimental import pallas as pl
from jax.experimental.pallas import tpu as pltpu
from jax.experimental.pallas import tpu_sc as plsc

D = 128
NPOS, NPC, NCOL = 64, 6, 2
NFUSED = NPOS * NPC * NCOL

B, L = 16384, 32
TOK = B * L
NC, NS = 2, 16
NW = NC * NS
TPW = TOK // NW
CHUNK = 128
NCHUNK = TPW // CHUNK
ROWS = TOK // D
NBUF = 4


def _fuse_tables_kernel(pos_ref, pc_ref, col_ref, out_ref):
    pos = pos_ref[:]
    for m in range(NPC * NCOL):
        p, c = m // NCOL, m % NCOL
        out_ref[pl.ds(m * NPOS, NPOS), :] = (
            pos + pc_ref[pl.ds(p, 1), :] + col_ref[pl.ds(c, 1), :]
        )


def _fuse_tables(pos, pc, col):
    return pl.pallas_call(
        _fuse_tables_kernel,
        out_shape=jax.ShapeDtypeStruct((NFUSED, D), jnp.float32),
    )(pos, pc, col)


def _fidx_kernel(p_ref, c_ref, i_ref, o_ref):
    o_ref[:] = (p_ref[:] * NCOL + c_ref[:]) * NPOS + i_ref[:]


def _fused_index(p2, c2, i2):
    grid = 8
    blk = ROWS // grid
    return pl.pallas_call(
        _fidx_kernel,
        grid=(grid,),
        in_specs=[pl.BlockSpec((blk, D), lambda g: (g, 0))] * 3,
        out_specs=pl.BlockSpec((blk, D), lambda g: (g, 0)),
        out_shape=jax.ShapeDtypeStruct((ROWS, D), jnp.int32),
    )(p2, c2, i2)


def _sc_gather_body(tab_hbm, fidx_hbm, out_hbm, tab_sh, fidx_v, rows, semg, semo):
    sid = lax.axis_index("s")
    wid = sid * NC + lax.axis_index("c")
    row0 = wid * (TPW // D)
    tok0 = wid * TPW

    @pl.when(sid == 0)
    def _load_tab():
        pltpu.sync_copy(tab_hbm, tab_sh)

    pltpu.sync_copy(fidx_hbm.at[pl.ds(row0, TPW // D)], fidx_v)
    plsc.subcore_barrier()

    def gather(g, b):
        pltpu.async_copy(tab_sh.at[fidx_v.at[g]], rows[b], semg[b])

    def put(g, b):
        pltpu.async_copy(rows[b], out_hbm.at[pl.ds(tok0 + g * CHUNK, CHUNK)],
                         semo[b])

    def wait_gather(g, b):
        pltpu.make_async_copy(tab_sh.at[fidx_v.at[g]], rows[b], semg[b]).wait()

    def wait_put(g, b):
        pltpu.make_async_copy(rows[b],
                              out_hbm.at[pl.ds(tok0 + g * CHUNK, CHUNK)],
                              semo[b]).wait()

    gather(0, 0)
    gather(1, 1)

    def step(h, _):
        for b in range(NBUF):
            g = h * NBUF + b
            g2 = g + 2
            b2 = (b + 2) % NBUF

            @pl.when(g2 <= NCHUNK - 1)
            def _issue():
                @pl.when(g >= 2)
                def _free():
                    wait_put(g - 2, b2)
                gather(g2, b2)

            wait_gather(g, b)
            put(g, b)
        return _

    lax.fori_loop(0, NCHUNK // NBUF, step, None)

    for b in range(NBUF):
        wait_put(NCHUNK - NBUF + b, b)


def _sc_gather(tab, fidx2):
    mesh = plsc.VectorSubcoreMesh(core_axis_name="c", subcore_axis_name="s")
    f = functools.partial(
        pl.kernel,
        out_type=jax.ShapeDtypeStruct((TOK, D), jnp.float32),
        mesh=mesh,
        scratch_types=[
            pltpu.VMEM_SHARED((NFUSED, D), jnp.float32),
            pltpu.VMEM((TPW // D, D), jnp.int32),
            [pltpu.VMEM((CHUNK, D), jnp.float32) for _ in range(NBUF)],
            [pltpu.SemaphoreType.DMA for _ in range(NBUF)],
            [pltpu.SemaphoreType.DMA for _ in range(NBUF)],
        ],
    )(_sc_gather_body)
    return f(tab, fidx2)


def kernel(pieces_ids, color_ids, indexes, position_emb, piece_emb, color_emb):
    tab = _fuse_tables(position_emb, piece_emb, color_emb)
    p2 = pieces_ids.astype(jnp.int32).reshape(ROWS, D)
    c2 = color_ids.astype(jnp.int32).reshape(ROWS, D)
    i2 = indexes.astype(jnp.int32).reshape(ROWS, D)
    fidx2 = _fused_index(p2, c2, i2)
    out = _sc_gather(tab, fidx2)
    return out.reshape(B, L, D)


# speedup vs baseline: 37.1785x; 1.0032x over previous
import functools

import jax
import jax.numpy as jnp
from jax import lax
from jax.experimental import pallas as pl
from jax.experimental.pallas import tpu as pltpu
from jax.experimental.pallas import tpu_sc as plsc

D = 128
NPOS, NPC, NCOL = 64, 6, 2
NFUSED = NPOS * NPC * NCOL

B, L = 16384, 32
TOK = B * L
NC, NS = 2, 16
NW = NC * NS
TPW = TOK // NW
CHUNK = 128
NCHUNK = TPW // CHUNK
ROWS = TOK // D
RPW = TPW // D
NBUF = 4
LANES = 16


def _sc_body(pos_hbm, pc_hbm, col_hbm, p_hbm, c_hbm, i_hbm, out_hbm,
             tab_sh, fidx_v, rows, stage_p, stage_c, pos_v, pcrow_v, colrow_v,
             semg, semo):
    sid = lax.axis_index("s")
    wid = sid * NC + lax.axis_index("c")
    row0 = wid * RPW
    tok0 = wid * TPW

    cp_p = pltpu.async_copy(p_hbm.at[pl.ds(row0, RPW)], stage_p, semg[0])
    cp_c = pltpu.async_copy(c_hbm.at[pl.ds(row0, RPW)], stage_c, semg[1])
    cp_i = pltpu.async_copy(i_hbm.at[pl.ds(row0, RPW)], fidx_v, semg[2])

    @pl.when(sid < NPC * NCOL)
    def _build():
        m = sid
        pltpu.sync_copy(pos_hbm, pos_v)
        pltpu.sync_copy(pc_hbm.at[pl.ds(m // NCOL, 1)], pcrow_v)
        pltpu.sync_copy(col_hbm.at[pl.ds(m % NCOL, 1)], colrow_v)

        def build_row(r, carry):
            for k in range(D // LANES):
                s = pl.ds(k * LANES, LANES)
                pos_v[r, s] = pos_v[r, s] + pcrow_v[0, s] + colrow_v[0, s]
            return carry

        lax.fori_loop(0, NPOS, build_row, None)
        pltpu.sync_copy(pos_v, tab_sh.at[pl.ds(m * NPOS, NPOS)])

    cp_p.wait()
    cp_c.wait()
    cp_i.wait()

    def fidx_row(r, carry):
        for k in range(D // LANES):
            s = pl.ds(k * LANES, LANES)
            fidx_v[r, s] = (stage_p[r, s] * NCOL + stage_c[r, s]) * NPOS \
                + fidx_v[r, s]
        return carry

    lax.fori_loop(0, RPW, fidx_row, None)

    plsc.subcore_barrier()

    def gather(g, b):
        pltpu.async_copy(tab_sh.at[fidx_v.at[g]], rows[b], semg[b])

    def put(g, b):
        pltpu.async_copy(rows[b], out_hbm.at[pl.ds(tok0 + g * CHUNK, CHUNK)],
                         semo[b])

    def wait_gather(g, b):
        pltpu.make_async_copy(tab_sh.at[fidx_v.at[g]], rows[b], semg[b]).wait()

    def wait_put(g, b):
        pltpu.make_async_copy(rows[b],
                              out_hbm.at[pl.ds(tok0 + g * CHUNK, CHUNK)],
                              semo[b]).wait()

    gather(0, 0)
    gather(1, 1)

    def step(h, carry):
        for b in range(NBUF):
            g = h * NBUF + b
            g2 = g + 2
            b2 = (b + 2) % NBUF

            @pl.when(g2 <= NCHUNK - 1)
            def _issue():
                @pl.when(g >= 2)
                def _free():
                    wait_put(g - 2, b2)
                gather(g2, b2)

            wait_gather(g, b)
            put(g, b)
        return carry

    lax.fori_loop(0, NCHUNK // NBUF, step, None)

    for b in range(NBUF):
        wait_put(NCHUNK - NBUF + b, b)


def _sc_embed(pos, pc, col, p2f, c2f, i2f):
    mesh = plsc.VectorSubcoreMesh(core_axis_name="c", subcore_axis_name="s")
    f = functools.partial(
        pl.kernel,
        out_type=jax.ShapeDtypeStruct((TOK, D), jnp.float32),
        mesh=mesh,
        scratch_types=[
            pltpu.VMEM_SHARED((NFUSED, D), jnp.float32),
            pltpu.VMEM((RPW, D), jnp.int32),
            [pltpu.VMEM((CHUNK, D), jnp.float32) for _ in range(NBUF)],
            pltpu.VMEM((RPW, D), jnp.int32),
            pltpu.VMEM((RPW, D), jnp.int32),
            pltpu.VMEM((NPOS, D), jnp.float32),
            pltpu.VMEM((1, D), jnp.float32),
            pltpu.VMEM((1, D), jnp.float32),
            [pltpu.SemaphoreType.DMA for _ in range(NBUF)],
            [pltpu.SemaphoreType.DMA for _ in range(NBUF)],
        ],
    )(_sc_body)
    return f(pos, pc, col, p2f, c2f, i2f)


def kernel(pieces_ids, color_ids, indexes, position_emb, piece_emb, color_emb):
    p2 = pieces_ids.astype(jnp.int32).reshape(ROWS, D)
    c2 = color_ids.astype(jnp.int32).reshape(ROWS, D)
    i2 = indexes.astype(jnp.int32).reshape(ROWS, D)
    out = _sc_embed(position_emb, piece_emb, color_emb, p2, c2, i2)
    return out.reshape(B, L, D)


# speedup vs baseline: 44.3501x; 1.1929x over previous
import functools

import jax
import jax.numpy as jnp
from jax import lax
from jax.experimental import pallas as pl
from jax.experimental.pallas import tpu as pltpu
from jax.experimental.pallas import tpu_sc as plsc

D = 128
NPOS, NPC, NCOL = 64, 6, 2
NFUSED = NPOS * NPC * NCOL

B, L = 16384, 32
TOK = B * L
NC, NS = 2, 16
NW = NC * NS
TPW = TOK // NW
CHUNK = 128
NCHUNK = TPW // CHUNK
ROWS = TOK // D
RPW = TPW // D
NBUF = 4
LANES = 16


def _sc_body(pos_hbm, pc_hbm, col_hbm, fidx_hbm, out_hbm,
             tab_sh, fidx_v, rows, pos_v, pcrow_v, colrow_v,
             semg, semo):
    sid = lax.axis_index("s")
    wid = sid * NC + lax.axis_index("c")
    tok0 = wid * TPW
    row0 = wid * RPW

    cp_f = pltpu.async_copy(fidx_hbm.at[pl.ds(row0, RPW)], fidx_v, semg[0])

    @pl.when(sid < NPC * NCOL)
    def _build():
        m = sid
        pltpu.sync_copy(pos_hbm, pos_v)
        pltpu.sync_copy(pc_hbm.at[pl.ds(m // NCOL, 1)], pcrow_v)
        pltpu.sync_copy(col_hbm.at[pl.ds(m % NCOL, 1)], colrow_v)
        for k in range(D // LANES):
            s = pl.ds(k * LANES, LANES)
            pcrow_v[0, s] = pcrow_v[0, s] + colrow_v[0, s]

        def build_row(r, carry):
            for k in range(D // LANES):
                s = pl.ds(k * LANES, LANES)
                pos_v[r, s] = pos_v[r, s] + pcrow_v[0, s]
            return carry

        lax.fori_loop(0, NPOS, build_row, None)
        pltpu.sync_copy(pos_v, tab_sh.at[pl.ds(m * NPOS, NPOS)])

    cp_f.wait()
    plsc.subcore_barrier()

    def gather(g, b):
        pltpu.async_copy(tab_sh.at[fidx_v.at[g]], rows[b], semg[b])

    def put(g, b):
        pltpu.async_copy(rows[b], out_hbm.at[pl.ds(tok0 + g * CHUNK, CHUNK)],
                         semo[b])

    def wait_gather(g, b):
        pltpu.make_async_copy(tab_sh.at[fidx_v.at[g]], rows[b], semg[b]).wait()

    def wait_put(g, b):
        pltpu.make_async_copy(rows[b],
                              out_hbm.at[pl.ds(tok0 + g * CHUNK, CHUNK)],
                              semo[b]).wait()

    gather(0, 0)
    gather(1, 1)

    for b in range(NBUF):
        if b >= 2:
            wait_put(b - 2, b - 2)
        gather(b + 2, (b + 2) % NBUF)
        wait_gather(b, b)
        put(b, b)

    def step(h, carry):
        for b in range(NBUF):
            g = h * NBUF + b
            b2 = (b + 2) % NBUF
            wait_put(g - 2, b2)
            gather(g + 2, b2)
            wait_gather(g, b)
            put(g, b)
        return carry

    lax.fori_loop(1, NCHUNK // NBUF - 1, step, None)

    for b in range(NBUF):
        g = NCHUNK - NBUF + b
        if b < 2:
            wait_put(g - 2, (b + 2) % NBUF)
            gather(g + 2, (b + 2) % NBUF)
        wait_gather(g, b)
        put(g, b)

    for b in range(NBUF):
        wait_put(NCHUNK - NBUF + b, b)


def _sc_embed(pos, pc, col, fidx2):
    mesh = plsc.VectorSubcoreMesh(core_axis_name="c", subcore_axis_name="s")
    f = functools.partial(
        pl.kernel,
        out_type=jax.ShapeDtypeStruct((TOK, D), jnp.float32),
        mesh=mesh,
        scratch_types=[
            pltpu.VMEM_SHARED((NFUSED, D), jnp.float32),
            pltpu.VMEM((RPW, D), jnp.int32),
            [pltpu.VMEM((CHUNK, D), jnp.float32) for _ in range(NBUF)],
            pltpu.VMEM((NPOS, D), jnp.float32),
            pltpu.VMEM((1, D), jnp.float32),
            pltpu.VMEM((1, D), jnp.float32),
            [pltpu.SemaphoreType.DMA for _ in range(NBUF)],
            [pltpu.SemaphoreType.DMA for _ in range(NBUF)],
        ],
    )(_sc_body)
    return f(pos, pc, col, fidx2)


def kernel(pieces_ids, color_ids, indexes, position_emb, piece_emb, color_emb):
    fidx = (pieces_ids.astype(jnp.int32) * NCOL
            + color_ids.astype(jnp.int32)) * NPOS + indexes.astype(jnp.int32)
    out = _sc_embed(position_emb, piece_emb, color_emb, fidx.reshape(ROWS, D))
    return out.reshape(B, L, D)
